# baseline (speedup 1.0000x reference)
import functools

import jax
import jax.numpy as jnp
from jax.experimental import pallas as pl


def _gelu_exact(v):
    return 0.5 * v * (1.0 + jax.lax.erf(v * jnp.float32(0.7071067811865476)))


def _body(nsteps, n_tokens, x_ref, w1_ref, b1_ref, w2_ref, b2_ref, wd_ref,
          rmat_ref, sel_ref, wup_ref, out_ref, aux_ref, psum_ref, lsum_ref):
    step = pl.program_id(0)
    xb = x_ref[0].T
    E = rmat_ref.shape[1]

    xb_h = xb.astype(jnp.bfloat16)
    h = _gelu_exact(
        jnp.dot(xb_h, w1_ref[...], preferred_element_type=jnp.float32)
        + b1_ref[...])
    shared = (jnp.dot(h.astype(jnp.bfloat16), w2_ref[...],
                      preferred_element_type=jnp.float32)
              + b2_ref[...])

    feats = jnp.dot(xb, wd_ref[...], preferred_element_type=jnp.float32)
    logits = jnp.dot(feats, rmat_ref[...], preferred_element_type=jnp.float32)

    m = jnp.max(logits, axis=1, keepdims=True)
    ex = jnp.exp(logits - m)
    probs = ex / jnp.sum(ex, axis=1, keepdims=True)

    eidx = jax.lax.broadcasted_iota(jnp.int32, probs.shape, 1)
    m1 = jnp.max(probs, axis=1, keepdims=True)
    i1 = jnp.min(jnp.where(probs == m1, eidx, E), axis=1, keepdims=True)
    mask1 = eidx == i1
    rest = jnp.where(mask1, -1.0, probs)
    m2 = jnp.max(rest, axis=1, keepdims=True)
    i2 = jnp.min(jnp.where(rest == m2, eidx, E), axis=1, keepdims=True)
    sel = mask1 | (eidx == i2)
    gate = jnp.where(sel, probs, 0.0) / (m1 + m2)

    gate_big = jnp.dot(gate, sel_ref[...],
                       preferred_element_type=jnp.float32)
    wf = _gelu_exact(feats) * gate_big
    aoe = jnp.dot(wf.astype(jnp.bfloat16), wup_ref[...],
                  preferred_element_type=jnp.float32)

    out_ref[0] = (xb + shared + aoe).T

    p_part = jnp.sum(probs, axis=0, keepdims=True)
    l_part = jnp.sum(sel.astype(jnp.float32), axis=0, keepdims=True)

    @pl.when(step == 0)
    def _init():
        psum_ref[...] = jnp.zeros_like(psum_ref)
        lsum_ref[...] = jnp.zeros_like(lsum_ref)

    psum_ref[...] += p_part
    lsum_ref[...] += l_part

    @pl.when(step == nsteps - 1)
    def _fin():
        n_f = jnp.float32(n_tokens)
        aux_ref[...] = (jnp.float32(E) / (n_f * n_f)
                        * jnp.sum(psum_ref[...] * lsum_ref[...], keepdims=True))


def kernel(x, conv1_w, conv1_b, conv2_w, conv2_b, w_down, router_w, w_up):
    B, C, H, W = x.shape
    E, d_low, _ = w_up.shape
    hid = conv1_w.shape[0]
    HW = H * W
    N = B * HW

    x3 = x.reshape(B, C, HW)
    w1t = conv1_w.T.astype(jnp.bfloat16)
    w2t = conv2_w.T.astype(jnp.bfloat16)
    wdt = w_down.T
    eye = jnp.eye(E, dtype=x.dtype)
    rmat = jnp.kron(eye, router_w[0][:, None])
    selm = jnp.kron(eye, jnp.ones((1, d_low), x.dtype))
    wupf = w_up.reshape(E * d_low, C).astype(jnp.bfloat16)

    full = lambda r, c: pl.BlockSpec((r, c), lambda i: (0, 0))
    out3, aux, _, _ = pl.pallas_call(
        functools.partial(_body, B, N),
        grid=(B,),
        in_specs=[
            pl.BlockSpec((1, C, HW), lambda i: (i, 0, 0)),
            full(C, hid), full(1, hid), full(hid, C), full(1, C),
            full(C, E * d_low), full(E * d_low, E), full(E, E * d_low),
            full(E * d_low, C),
        ],
        out_specs=[
            pl.BlockSpec((1, C, HW), lambda i: (i, 0, 0)),
            full(1, 1), full(1, E), full(1, E),
        ],
        out_shape=[
            jax.ShapeDtypeStruct((B, C, HW), jnp.float32),
            jax.ShapeDtypeStruct((1, 1), jnp.float32),
            jax.ShapeDtypeStruct((1, E), jnp.float32),
            jax.ShapeDtypeStruct((1, E), jnp.float32),
        ],
    )(x3, w1t, conv1_b[None, :], w2t, conv2_b[None, :], wdt, rmat, selm,
      wupf)

    return (out3.reshape(B, C, H, W), aux[0, 0])


# speedup vs baseline: 1.3893x; 1.3893x over previous
import functools

import jax
import jax.numpy as jnp
from jax.experimental import pallas as pl

_TN = 512

_DN_NT = (((1,), (1,)), ((), ()))


def _gelu_exact(v):
    return 0.5 * v * (1.0 + jax.lax.erf(v * jnp.float32(0.7071067811865476)))


def _body(nsteps, n_tokens, x_ref, w1_ref, b1_ref, w2_ref, b2_ref, wd_ref,
          rmat_ref, sel_ref, wup_ref, out_ref, aux_ref, psum_ref, lsum_ref):
    step = pl.program_id(0)
    xb = x_ref[...]
    E = rmat_ref.shape[1]

    xb_h = xb.astype(jnp.bfloat16)
    h = _gelu_exact(
        jax.lax.dot_general(xb_h, w1_ref[...], _DN_NT,
                            preferred_element_type=jnp.float32)
        + b1_ref[...])
    shared = (jax.lax.dot_general(h.astype(jnp.bfloat16), w2_ref[...], _DN_NT,
                                  preferred_element_type=jnp.float32)
              + b2_ref[...])

    feats = jax.lax.dot_general(xb, wd_ref[...], _DN_NT,
                                preferred_element_type=jnp.float32)
    logits = jnp.dot(feats, rmat_ref[...], preferred_element_type=jnp.float32)

    m = jnp.max(logits, axis=1, keepdims=True)
    ex = jnp.exp(logits - m)
    probs = ex / jnp.sum(ex, axis=1, keepdims=True)

    eidx = jax.lax.broadcasted_iota(jnp.int32, probs.shape, 1)
    m1 = jnp.max(probs, axis=1, keepdims=True)
    i1 = jnp.min(jnp.where(probs == m1, eidx, E), axis=1, keepdims=True)
    mask1 = eidx == i1
    rest = jnp.where(mask1, -1.0, probs)
    m2 = jnp.max(rest, axis=1, keepdims=True)
    i2 = jnp.min(jnp.where(rest == m2, eidx, E), axis=1, keepdims=True)
    sel = mask1 | (eidx == i2)
    gate = jnp.where(sel, probs, 0.0) / (m1 + m2)

    gate_big = jnp.dot(gate, sel_ref[...],
                       preferred_element_type=jnp.float32)
    wf = _gelu_exact(feats) * gate_big
    aoe = jnp.dot(wf.astype(jnp.bfloat16), wup_ref[...],
                  preferred_element_type=jnp.float32)

    out_ref[...] = xb + shared + aoe

    p_part = jnp.sum(probs, axis=0, keepdims=True)
    l_part = jnp.sum(sel.astype(jnp.float32), axis=0, keepdims=True)

    @pl.when(step == 0)
    def _init():
        psum_ref[...] = jnp.zeros_like(psum_ref)
        lsum_ref[...] = jnp.zeros_like(lsum_ref)

    psum_ref[...] += p_part
    lsum_ref[...] += l_part

    @pl.when(step == nsteps - 1)
    def _fin():
        n_f = jnp.float32(n_tokens)
        aux_ref[...] = (jnp.float32(E) / (n_f * n_f)
                        * jnp.sum(psum_ref[...] * lsum_ref[...], keepdims=True))


def kernel(x, conv1_w, conv1_b, conv2_w, conv2_b, w_down, router_w, w_up):
    B, C, H, W = x.shape
    E, d_low, _ = w_up.shape
    hid = conv1_w.shape[0]
    N = B * H * W
    nsteps = N // _TN
    assert N % _TN == 0

    x_tok = x.transpose(0, 2, 3, 1).reshape(N, C)
    w1b = conv1_w.astype(jnp.bfloat16)
    w2b = conv2_w.astype(jnp.bfloat16)
    eye = jnp.eye(E, dtype=x.dtype)
    rmat = jnp.kron(eye, router_w[0][:, None])
    selm = jnp.kron(eye, jnp.ones((1, d_low), x.dtype))
    wupf = w_up.reshape(E * d_low, C).astype(jnp.bfloat16)

    full = lambda r, c: pl.BlockSpec((r, c), lambda i: (0, 0))
    out_tok, aux, _, _ = pl.pallas_call(
        functools.partial(_body, nsteps, N),
        grid=(nsteps,),
        in_specs=[
            pl.BlockSpec((_TN, C), lambda i: (i, 0)),
            full(hid, C), full(1, hid), full(C, hid), full(1, C),
            full(E * d_low, C), full(E * d_low, E), full(E, E * d_low),
            full(E * d_low, C),
        ],
        out_specs=[
            pl.BlockSpec((_TN, C), lambda i: (i, 0)),
            full(1, 1), full(1, E), full(1, E),
        ],
        out_shape=[
            jax.ShapeDtypeStruct((N, C), jnp.float32),
            jax.ShapeDtypeStruct((1, 1), jnp.float32),
            jax.ShapeDtypeStruct((1, E), jnp.float32),
            jax.ShapeDtypeStruct((1, E), jnp.float32),
        ],
    )(x_tok, w1b, conv1_b[None, :], w2b, conv2_b[None, :], w_down, rmat, selm,
      wupf)

    out = out_tok.reshape(B, H, W, C).transpose(0, 3, 1, 2)
    return (out, aux[0, 0])
